# Initial kernel scaffold; baseline (speedup 1.0000x reference)
import functools

import jax
import jax.numpy as jnp
from jax import lax
from jax.experimental import pallas as pl
from jax.experimental.pallas import tpu as pltpu
from jax.experimental.pallas import tpu_sc as plsc

_N = 10000
_E = 160000
_F_IN = 256
_F_OUT = 256
_H = 128
_NC = 2
_NS = 16
_EPT = _E // _NS
_K = 80
_NCH = _EPT // _K
_RPT = _N // _NS
_LANES = 16


def _matmul_body(x_ref, w_ref, o_ref):
    o_ref[...] = jnp.dot(x_ref[...], w_ref[...],
                         preferred_element_type=jnp.float32)


def _matmul(x, w):
    return pl.pallas_call(
        _matmul_body,
        grid=(_NC,),
        in_specs=[
            pl.BlockSpec((_N, _F_IN), lambda n: (0, 0)),
            pl.BlockSpec((_F_IN, _H), lambda n: (0, n)),
        ],
        out_specs=pl.BlockSpec((_N, _H), lambda n: (n, 0)),
        out_shape=jax.ShapeDtypeStruct((_NC * _N, _H), jnp.float32),
    )(x, w)


def _spmm_body(xws, colr, rowr, adjr, zeros, out,
               col_v, row_v, adj_v, rows_v, acc, sem):
    c = lax.axis_index("c")
    s = lax.axis_index("s")
    w = c * _NS + s
    r0 = s * _RPT

    pltpu.sync_copy(zeros, acc.at[pl.ds(r0, _RPT)])
    pltpu.sync_copy(colr.at[w], col_v)
    pltpu.sync_copy(rowr.at[s], row_v)
    pltpu.sync_copy(adjr.at[s], adj_v)
    plsc.subcore_barrier()

    def chunk_body(j, carry):
        pltpu.async_copy(xws.at[col_v.at[j]], rows_v, sem).wait()

        def edge_body(i, carry2):
            ji = jnp.full((_LANES,), j, dtype=jnp.int32)
            ii = jnp.full((_LANES,), i, dtype=jnp.int32)
            a = plsc.load_gather(adj_v, [ji, ii])
            for f in range(_H // _LANES):
                seg = rows_v[i, pl.ds(f * _LANES, _LANES)]
                rows_v[i, pl.ds(f * _LANES, _LANES)] = seg * a
            return carry2

        lax.fori_loop(0, _K, edge_body, 0)

        pltpu.sync_copy(rows_v, acc.at[row_v.at[j]], add=True)
        return carry

    lax.fori_loop(0, _NCH, chunk_body, 0)
    plsc.subcore_barrier()

    pltpu.sync_copy(acc.at[pl.ds(r0, _RPT)],
                    out.at[pl.ds(c * _N + r0, _RPT)])


_spmm = functools.partial(
    pl.kernel,
    out_type=jax.ShapeDtypeStruct((_NC * _N, _H), jnp.float32),
    mesh=plsc.VectorSubcoreMesh(core_axis_name="c", subcore_axis_name="s"),
    scratch_types=[
        pltpu.VMEM((_NCH, _K), jnp.int32),
        pltpu.VMEM((_NCH, _K), jnp.int32),
        pltpu.VMEM((_NCH, _K), jnp.float32),
        pltpu.VMEM((_K, _H), jnp.float32),
        pltpu.VMEM_SHARED((_N, _H), jnp.float32),
        pltpu.SemaphoreType.DMA,
    ],
)(_spmm_body)


def kernel(x, edge_index, adj_values, W, bias):
    row = edge_index[0]
    col = edge_index[1]

    xws = _matmul(x, W)

    colr = jnp.stack([col, col + _N]).reshape(_NC * _NS, _NCH, _K)
    rowr = row.reshape(_NS, _NCH, _K)
    adjr = adj_values.reshape(_NS, _NCH, _K)
    zeros = jnp.zeros((_RPT, _H), dtype=jnp.float32)

    outs = _spmm(xws, colr, rowr, adjr, zeros)

    out = outs.reshape(_NC, _N, _H).transpose(1, 0, 2).reshape(_N, _F_OUT)
    return out + bias


# speedup vs baseline: 4.3778x; 4.3778x over previous
import functools

import jax
import jax.numpy as jnp
from jax import lax
from jax.experimental import pallas as pl
from jax.experimental.pallas import tpu as pltpu
from jax.experimental.pallas import tpu_sc as plsc

_N = 10000
_E = 160000
_F_IN = 256
_F_OUT = 256
_H = 128
_NC = 2
_NS = 16
_EPT = _E // _NS
_K = 80
_NCH = _EPT // _K
_RPT = 624
_LANES = 16

_BCAST_DNUMS = lax.GatherDimensionNumbers(
    offset_dims=(), collapsed_slice_dims=(0,), start_index_map=(0,))


def _matmul_body(x_ref, w_ref, o_ref):
    o_ref[...] = jnp.dot(x_ref[...], w_ref[...],
                         preferred_element_type=jnp.float32)


def _matmul(x, w):
    return pl.pallas_call(
        _matmul_body,
        grid=(_NC,),
        in_specs=[
            pl.BlockSpec((_N, _F_IN), lambda n: (0, 0)),
            pl.BlockSpec((_F_IN, _H), lambda n: (0, n)),
        ],
        out_specs=pl.BlockSpec((_N, _H), lambda n: (n, 0)),
        out_shape=jax.ShapeDtypeStruct((_NC * _N, _H), jnp.float32),
    )(x, w)


def _spmm_body(xws, colr, rowr, adjr, zeros, out,
               col_v, row_v, adj_v, rows_v, acc, sem):
    c = lax.axis_index("c")
    s = lax.axis_index("s")
    w = c * _NS + s
    r0 = s * _RPT

    pltpu.sync_copy(zeros.at[pl.ds(0, _RPT)], acc.at[pl.ds(r0, _RPT)])

    @pl.when(s == _NS - 1)
    def _():
        rem = _N - _NS * _RPT
        pltpu.sync_copy(zeros.at[pl.ds(0, rem)],
                        acc.at[pl.ds(_NS * _RPT, rem)])
    pltpu.sync_copy(colr.at[pl.ds(w * _EPT, _EPT)], col_v)
    pltpu.sync_copy(rowr.at[s], row_v)
    pltpu.sync_copy(adjr.at[pl.ds(s * _EPT, _EPT)], adj_v)
    plsc.subcore_barrier()

    def chunk_body(j, carry):
        pltpu.async_copy(xws.at[col_v.at[pl.ds(j * _K, _K)]],
                         rows_v, sem).wait()

        def group_body(g, carry2):
            av = adj_v[pl.ds(j * _K + g * _LANES, _LANES)]
            for t in range(_LANES):
                a = lax.gather(
                    av,
                    jnp.full((_LANES, 1), t, dtype=jnp.int32),
                    _BCAST_DNUMS,
                    slice_sizes=(1,),
                    mode=lax.GatherScatterMode.PROMISE_IN_BOUNDS,
                )
                e = g * _LANES + t
                for f in range(_H // _LANES):
                    seg = rows_v[e, pl.ds(f * _LANES, _LANES)]
                    rows_v[e, pl.ds(f * _LANES, _LANES)] = seg * a
            return carry2

        lax.fori_loop(0, _K // _LANES, group_body, 0)

        pltpu.sync_copy(rows_v, acc.at[row_v.at[j]], add=True)
        return carry

    lax.fori_loop(0, _NCH, chunk_body, 0)
    plsc.subcore_barrier()

    pltpu.sync_copy(acc.at[pl.ds(r0, _RPT)],
                    out.at[pl.ds(c * _N + r0, _RPT)])

    @pl.when(s == _NS - 1)
    def _():
        rem = _N - _NS * _RPT
        pltpu.sync_copy(acc.at[pl.ds(_NS * _RPT, rem)],
                        out.at[pl.ds(c * _N + _NS * _RPT, rem)])


_spmm = functools.partial(
    pl.kernel,
    out_type=jax.ShapeDtypeStruct((_NC * _N, _H), jnp.float32),
    mesh=plsc.VectorSubcoreMesh(core_axis_name="c", subcore_axis_name="s"),
    scratch_types=[
        pltpu.VMEM((_EPT,), jnp.int32),
        pltpu.VMEM((_NCH, _K), jnp.int32),
        pltpu.VMEM((_EPT,), jnp.float32),
        pltpu.VMEM((_K, _H), jnp.float32),
        pltpu.VMEM_SHARED((_N, _H), jnp.float32),
        pltpu.SemaphoreType.DMA,
    ],
)(_spmm_body)


def kernel(x, edge_index, adj_values, W, bias):
    row = edge_index[0]
    col = edge_index[1]

    xws = _matmul(x, W)

    colr = jnp.concatenate([col, col + _N])
    rowr = row.reshape(_NS, _NCH, _K)
    adjr = adj_values
    zeros = jnp.zeros((_RPT + 16, _H), dtype=jnp.float32)

    outs = _spmm(xws, colr, rowr, adjr, zeros)

    out = outs.reshape(_NC, _N, _H).transpose(1, 0, 2).reshape(_N, _F_OUT)
    return out + bias
